# Initial kernel scaffold; baseline (speedup 1.0000x reference)
import jax
import jax.numpy as jnp
from jax.experimental import pallas as pl


def kernel(x, tables):
    raise NotImplementedError("write your pallas kernel here")


# speedup vs baseline: 10.8044x; 10.8044x over previous
import functools

import jax
import jax.numpy as jnp
from jax import lax
from jax.experimental import pallas as pl
from jax.experimental.pallas import tpu as pltpu
from jax.experimental.pallas import tpu_sc as plsc

NUM_TOKENS = 16
VOCAB = 100000
SPLIT_DIM = 8

NUM_CORES = 2
NUM_SUBCORES = 16
NUM_WORKERS = NUM_CORES * NUM_SUBCORES
LANES = 16

CHUNK = 2048


def _make_lookup(n, d):
    per_w = n // NUM_WORKERS
    n_chunks = per_w // CHUNK
    assert per_w % CHUNK == 0 and n % NUM_WORKERS == 0

    mesh = plsc.VectorSubcoreMesh(core_axis_name="c", subcore_axis_name="s")

    @functools.partial(
        pl.kernel,
        mesh=mesh,
        compiler_params=pltpu.CompilerParams(use_tc_tiling_on_sc=False),
        out_type=jax.ShapeDtypeStruct((n, d), jnp.float32),
        scratch_types=[
            pltpu.VMEM((CHUNK,), jnp.int32),
            pltpu.VMEM((CHUNK, d), jnp.float32),
            pltpu.SemaphoreType.DMA,
        ],
    )
    def lookup(x_hbm, tab_hbm, out_hbm, idx_v, rows_v, sem):
        wid = lax.axis_index("s") * NUM_CORES + lax.axis_index("c")
        base_w = wid * per_w
        offs = lax.iota(jnp.int32, LANES) * VOCAB

        def chunk_body(k, carry):
            base = base_w + k * CHUNK
            pltpu.sync_copy(x_hbm.at[pl.ds(base, CHUNK)], idx_v)

            def add_off(j, c):
                sl = pl.ds(j * LANES, LANES)
                idx_v[sl] = idx_v[sl] + offs
                return c

            lax.fori_loop(0, CHUNK // LANES, add_off, 0)
            pltpu.async_copy(tab_hbm.at[idx_v], rows_v, sem).wait()
            pltpu.sync_copy(rows_v, out_hbm.at[pl.ds(base, CHUNK)])
            return carry

        lax.fori_loop(0, n_chunks, chunk_body, 0)

    return lookup


def kernel(x, tables):
    batch, seq, num_tok = x.shape
    t, vocab, d = tables.shape
    n = batch * seq * num_tok
    x_flat = x.reshape(n)
    tab_flat = tables.reshape(t * vocab, d)
    out = _make_lookup(n, d)(x_flat, tab_flat)
    return out.reshape(batch, seq, num_tok * d)


# speedup vs baseline: 11.8864x; 1.1001x over previous
import functools

import jax
import jax.numpy as jnp
from jax import lax
from jax.experimental import pallas as pl
from jax.experimental.pallas import tpu as pltpu
from jax.experimental.pallas import tpu_sc as plsc

NUM_TOKENS = 16
VOCAB = 100000
SPLIT_DIM = 8

NUM_CORES = 2
NUM_SUBCORES = 16
NUM_WORKERS = NUM_CORES * NUM_SUBCORES
LANES = 16

CHUNK = 4096
ADD_UNROLL = 4


def _make_lookup(n, d):
    per_w = n // NUM_WORKERS
    n_chunks = per_w // CHUNK
    assert n % NUM_WORKERS == 0 and per_w % CHUNK == 0
    assert n_chunks % 2 == 1

    mesh = plsc.VectorSubcoreMesh(core_axis_name="c", subcore_axis_name="s")

    @functools.partial(
        pl.kernel,
        mesh=mesh,
        compiler_params=pltpu.CompilerParams(use_tc_tiling_on_sc=False),
        out_type=jax.ShapeDtypeStruct((n, d), jnp.float32),
        scratch_types=[
            pltpu.VMEM((CHUNK,), jnp.int32),
            pltpu.VMEM((CHUNK,), jnp.int32),
            pltpu.VMEM((CHUNK, d), jnp.float32),
            pltpu.VMEM((CHUNK, d), jnp.float32),
            pltpu.SemaphoreType.DMA,
            pltpu.SemaphoreType.DMA,
        ],
    )
    def lookup(x_hbm, tab_hbm, out_hbm, idx0, idx1, rows0, rows1, s0, s1):
        wid = lax.axis_index("s") * NUM_CORES + lax.axis_index("c")
        base_w = wid * per_w
        offs = lax.iota(jnp.int32, LANES) * VOCAB

        def load_and_offset(k, idx_v):
            pltpu.sync_copy(x_hbm.at[pl.ds(base_w + k * CHUNK, CHUNK)], idx_v)

            def add_body(j, c):
                for u in range(ADD_UNROLL):
                    sl = pl.ds((j * ADD_UNROLL + u) * LANES, LANES)
                    idx_v[sl] = idx_v[sl] + offs
                return c

            lax.fori_loop(0, CHUNK // (LANES * ADD_UNROLL), add_body, 0)

        def store_out(k, rows_v):
            pltpu.sync_copy(rows_v, out_hbm.at[pl.ds(base_w + k * CHUNK, CHUNK)])

        load_and_offset(0, idx0)
        pltpu.async_copy(tab_hbm.at[idx0], rows0, s0)

        def pair_body(p, c):
            k0 = 2 * p
            load_and_offset(k0 + 1, idx1)
            pltpu.async_copy(tab_hbm.at[idx1], rows1, s1)
            pltpu.make_async_copy(tab_hbm.at[idx0], rows0, s0).wait()
            store_out(k0, rows0)
            load_and_offset(k0 + 2, idx0)
            pltpu.async_copy(tab_hbm.at[idx0], rows0, s0)
            pltpu.make_async_copy(tab_hbm.at[idx1], rows1, s1).wait()
            store_out(k0 + 1, rows1)
            return c

        lax.fori_loop(0, (n_chunks - 1) // 2, pair_body, 0)

        pltpu.make_async_copy(tab_hbm.at[idx0], rows0, s0).wait()
        store_out(n_chunks - 1, rows0)

    return lookup


def kernel(x, tables):
    batch, seq, num_tok = x.shape
    t, vocab, d = tables.shape
    n = batch * seq * num_tok
    x_flat = x.reshape(n)
    tab_flat = tables.reshape(t * vocab, d)
    out = _make_lookup(n, d)(x_flat, tab_flat)
    return out.reshape(batch, seq, num_tok * d)
